# baseline (speedup 1.0000x reference)
import jax
import jax.numpy as jnp
from jax.experimental import pallas as pl
from jax.experimental.pallas import tpu as pltpu

_HP, _WP = 12, 12
_NP = _HP * _WP
_KT = 36
_K2 = 4 * 4 * 32
_N2 = 2 * 2 * 64
_FEAT = _NP * 64
_BT = 64
_BM = 1024


def _conv_kernel(p_ref, w1e_ref, w2e_ref, b2e_ref, fw1_ref, fb1_ref,
                 fw2_ref, fb2_ref, o_ref, f_ref):
    bt = p_ref.shape[1]
    pt = p_ref[...].reshape(_KT, bt * _NP)
    ptb = jnp.concatenate(
        [pt, jnp.ones((1, bt * _NP), pt.dtype)], axis=0)
    x2 = jax.lax.dot_general(
        ptb, w1e_ref[...],
        dimension_numbers=(((0,), (0,)), ((), ())),
        preferred_element_type=jnp.float32)
    x2 = jnp.maximum(x2, 0.0).astype(jnp.bfloat16)
    o2 = jnp.dot(x2, w2e_ref[...], preferred_element_type=jnp.float32)
    m = jnp.maximum(jnp.maximum(o2[:, 0:64], o2[:, 64:128]),
                    jnp.maximum(o2[:, 128:192], o2[:, 192:256]))
    m = jnp.maximum(m + b2e_ref[...], 0.0)
    f_ref[...] = m.astype(jnp.bfloat16).reshape(bt, _NP, 64)
    feat = f_ref[...].reshape(bt, _FEAT)
    h = jnp.dot(feat, fw1_ref[...], preferred_element_type=jnp.float32)
    h = jnp.maximum(h + fb1_ref[...], 0.0)
    logits = jnp.dot(h, fw2_ref[...],
                     preferred_element_type=jnp.float32) + fb2_ref[...]
    mx = jnp.max(logits, axis=-1, keepdims=True)
    sh = logits - mx
    lse = jnp.log(jnp.sum(jnp.exp(sh), axis=-1, keepdims=True))
    o_ref[...] = (sh - lse).astype(o_ref.dtype)


def _conv_stage(p, w1e, w2e, b2, fw1, fb1, fw2, fb2):
    B = p.shape[1]
    bt = min(_BT, B)
    n = fw2.shape[1]
    return pl.pallas_call(
        _conv_kernel,
        out_shape=jax.ShapeDtypeStruct((B, n), jnp.float32),
        grid=(B // bt,),
        in_specs=[
            pl.BlockSpec((_KT, bt, _NP), lambda i: (0, i, 0)),
            pl.BlockSpec((_KT + 1, _K2), lambda i: (0, 0)),
            pl.BlockSpec((_K2, _N2), lambda i: (0, 0)),
            pl.BlockSpec((1, 64), lambda i: (0, 0)),
            pl.BlockSpec((_FEAT, 128), lambda i: (0, 0)),
            pl.BlockSpec((1, 128), lambda i: (0, 0)),
            pl.BlockSpec((128, n), lambda i: (0, 0)),
            pl.BlockSpec((1, n), lambda i: (0, 0)),
        ],
        out_specs=pl.BlockSpec((bt, n), lambda i: (i, 0)),
        scratch_shapes=[pltpu.VMEM((bt, _NP, 64), jnp.bfloat16)],
        compiler_params=pltpu.CompilerParams(
            dimension_semantics=("parallel",)),
    )(p, w1e, w2e, b2, fw1, fb1, fw2, fb2)


def _fc_kernel(x_ref, w1_ref, b1_ref, w2_ref, b2_ref, o_ref):
    h = jnp.dot(x_ref[...], w1_ref[...], preferred_element_type=jnp.float32)
    h = jnp.maximum(h + b1_ref[...], 0.0)
    logits = jnp.dot(h, w2_ref[...],
                     preferred_element_type=jnp.float32) + b2_ref[...]
    mx = jnp.max(logits, axis=-1, keepdims=True)
    s = logits - mx
    lse = jnp.log(jnp.sum(jnp.exp(s), axis=-1, keepdims=True))
    o_ref[...] = (s - lse).astype(o_ref.dtype)


def _fc_stage(feat, w1, b1, w2, b2):
    B = feat.shape[0]
    n = w2.shape[1]
    bm = min(_BM, B)
    return pl.pallas_call(
        _fc_kernel,
        out_shape=jax.ShapeDtypeStruct((B, n), jnp.float32),
        grid=(B // bm,),
        in_specs=[
            pl.BlockSpec((bm, _FEAT), lambda i: (i, 0)),
            pl.BlockSpec((_FEAT, 128), lambda i: (0, 0)),
            pl.BlockSpec((1, 128), lambda i: (0, 0)),
            pl.BlockSpec((128, n), lambda i: (0, 0)),
            pl.BlockSpec((1, n), lambda i: (0, 0)),
        ],
        out_specs=pl.BlockSpec((bm, n), lambda i: (i, 0)),
        compiler_params=pltpu.CompilerParams(
            dimension_semantics=("parallel",)),
    )(feat, w1, b1, w2, b2)


def _build_patches(x):
    B = x.shape[0]
    xf = x.reshape(B, 784).astype(jnp.bfloat16)
    di = jnp.arange(6).reshape(6, 1, 1, 1)
    dj = jnp.arange(6).reshape(1, 6, 1, 1)
    ph = jnp.arange(12).reshape(1, 1, 12, 1)
    pw = jnp.arange(12).reshape(1, 1, 1, 12)
    idx = ((2 * ph + di) * 28 + 2 * pw + dj).reshape(_KT * _NP)
    p = jnp.take(xf, idx, axis=1).reshape(B, _KT, _NP)
    return p.transpose(1, 0, 2)


def _build_w1e(w1m):
    di = jnp.arange(6).reshape(6, 1, 1, 1)
    dj = jnp.arange(6).reshape(1, 6, 1, 1)
    ei = jnp.arange(4).reshape(1, 1, 4, 1)
    ej = jnp.arange(4).reshape(1, 1, 1, 4)
    i1 = di - ei
    j1 = dj - ej
    valid = (i1 >= 0) & (i1 < 3) & (j1 >= 0) & (j1 < 3)
    idx = jnp.clip(i1, 0, 2) * 3 + jnp.clip(j1, 0, 2)
    w = w1m[idx] * valid[..., None].astype(w1m.dtype)
    return w.reshape(_KT, _K2)


def _build_w2e(w2m):
    ei = jnp.arange(4).reshape(4, 1, 1, 1)
    ej = jnp.arange(4).reshape(1, 4, 1, 1)
    dh = jnp.arange(2).reshape(1, 1, 2, 1)
    dw = jnp.arange(2).reshape(1, 1, 1, 2)
    i2 = ei - dh
    j2 = ej - dw
    valid = (i2 >= 0) & (i2 < 3) & (j2 >= 0) & (j2 < 3)
    idx = jnp.clip(i2, 0, 2) * 3 + jnp.clip(j2, 0, 2)
    w = w2m[idx]
    w = w * valid[..., None, None].astype(w2m.dtype)
    w = w.transpose(0, 1, 4, 2, 3, 5)
    return w.reshape(_K2, _N2)


def kernel(w1m, b1, w2m, b2, fc1_w, fc1_b, fc2_w, fc2_b, x):
    B = x.shape[0]
    p = _build_patches(x)
    b1e = jnp.broadcast_to(b1.reshape(1, 1, 32),
                           (16, 1, 32)).reshape(1, _K2)
    w1e = jnp.concatenate([_build_w1e(w1m), b1e],
                          axis=0).astype(jnp.bfloat16)
    w2e = _build_w2e(w2m).astype(jnp.bfloat16)
    return _conv_stage(p, w1e, w2e, b2.reshape(1, 64),
                       fc1_w.astype(jnp.bfloat16), fc1_b, fc2_w, fc2_b)


# speedup vs baseline: 1.0352x; 1.0352x over previous
import jax
import jax.numpy as jnp
from jax.experimental import pallas as pl
from jax.experimental.pallas import tpu as pltpu

_HP, _WP = 12, 12
_NP = _HP * _WP
_KT = 36
_K2 = 4 * 4 * 32
_N2 = 2 * 2 * 64
_FEAT = _NP * 64
_BT = 128
_BM = 1024


def _conv_kernel(p_ref, w1e_ref, w2e_ref, b2e_ref, fw1_ref, fb1_ref,
                 fw2_ref, fb2_ref, o_ref, f_ref):
    bt = p_ref.shape[1]
    pt = p_ref[...].reshape(_KT, bt * _NP)
    ptb = jnp.concatenate(
        [pt, jnp.ones((1, bt * _NP), pt.dtype)], axis=0)
    x2 = jax.lax.dot_general(
        ptb, w1e_ref[...],
        dimension_numbers=(((0,), (0,)), ((), ())),
        preferred_element_type=jnp.float32)
    x2 = jnp.maximum(x2, 0.0).astype(jnp.bfloat16)
    o2 = jnp.dot(x2, w2e_ref[...], preferred_element_type=jnp.float32)
    m = jnp.maximum(jnp.maximum(o2[:, 0:64], o2[:, 64:128]),
                    jnp.maximum(o2[:, 128:192], o2[:, 192:256]))
    m = jnp.maximum(m + b2e_ref[...], 0.0)
    f_ref[...] = m.astype(jnp.bfloat16).reshape(bt, _NP, 64)
    feat = f_ref[...].reshape(bt, _FEAT)
    h = jnp.dot(feat, fw1_ref[...], preferred_element_type=jnp.float32)
    h = jnp.maximum(h + fb1_ref[...], 0.0)
    logits = jnp.dot(h, fw2_ref[...],
                     preferred_element_type=jnp.float32) + fb2_ref[...]
    mx = jnp.max(logits, axis=-1, keepdims=True)
    sh = logits - mx
    lse = jnp.log(jnp.sum(jnp.exp(sh), axis=-1, keepdims=True))
    o_ref[...] = (sh - lse).astype(o_ref.dtype)


def _conv_stage(p, w1e, w2e, b2, fw1, fb1, fw2, fb2):
    B = p.shape[1]
    bt = min(_BT, B)
    n = fw2.shape[1]
    return pl.pallas_call(
        _conv_kernel,
        out_shape=jax.ShapeDtypeStruct((B, n), jnp.float32),
        grid=(B // bt,),
        in_specs=[
            pl.BlockSpec((_KT, bt, _NP), lambda i: (0, i, 0)),
            pl.BlockSpec((_KT + 1, _K2), lambda i: (0, 0)),
            pl.BlockSpec((_K2, _N2), lambda i: (0, 0)),
            pl.BlockSpec((1, 64), lambda i: (0, 0)),
            pl.BlockSpec((_FEAT, 128), lambda i: (0, 0)),
            pl.BlockSpec((1, 128), lambda i: (0, 0)),
            pl.BlockSpec((128, n), lambda i: (0, 0)),
            pl.BlockSpec((1, n), lambda i: (0, 0)),
        ],
        out_specs=pl.BlockSpec((bt, n), lambda i: (i, 0)),
        scratch_shapes=[pltpu.VMEM((bt, _NP, 64), jnp.bfloat16)],
        compiler_params=pltpu.CompilerParams(
            dimension_semantics=("parallel",)),
    )(p, w1e, w2e, b2, fw1, fb1, fw2, fb2)


def _fc_kernel(x_ref, w1_ref, b1_ref, w2_ref, b2_ref, o_ref):
    h = jnp.dot(x_ref[...], w1_ref[...], preferred_element_type=jnp.float32)
    h = jnp.maximum(h + b1_ref[...], 0.0)
    logits = jnp.dot(h, w2_ref[...],
                     preferred_element_type=jnp.float32) + b2_ref[...]
    mx = jnp.max(logits, axis=-1, keepdims=True)
    s = logits - mx
    lse = jnp.log(jnp.sum(jnp.exp(s), axis=-1, keepdims=True))
    o_ref[...] = (s - lse).astype(o_ref.dtype)


def _fc_stage(feat, w1, b1, w2, b2):
    B = feat.shape[0]
    n = w2.shape[1]
    bm = min(_BM, B)
    return pl.pallas_call(
        _fc_kernel,
        out_shape=jax.ShapeDtypeStruct((B, n), jnp.float32),
        grid=(B // bm,),
        in_specs=[
            pl.BlockSpec((bm, _FEAT), lambda i: (i, 0)),
            pl.BlockSpec((_FEAT, 128), lambda i: (0, 0)),
            pl.BlockSpec((1, 128), lambda i: (0, 0)),
            pl.BlockSpec((128, n), lambda i: (0, 0)),
            pl.BlockSpec((1, n), lambda i: (0, 0)),
        ],
        out_specs=pl.BlockSpec((bm, n), lambda i: (i, 0)),
        compiler_params=pltpu.CompilerParams(
            dimension_semantics=("parallel",)),
    )(feat, w1, b1, w2, b2)


def _build_patches(x):
    B = x.shape[0]
    xf = x.reshape(B, 784).astype(jnp.bfloat16)
    di = jnp.arange(6).reshape(6, 1, 1, 1)
    dj = jnp.arange(6).reshape(1, 6, 1, 1)
    ph = jnp.arange(12).reshape(1, 1, 12, 1)
    pw = jnp.arange(12).reshape(1, 1, 1, 12)
    idx = ((2 * ph + di) * 28 + 2 * pw + dj).reshape(_KT * _NP)
    p = jnp.take(xf, idx, axis=1).reshape(B, _KT, _NP)
    return p.transpose(1, 0, 2)


def _build_w1e(w1m):
    di = jnp.arange(6).reshape(6, 1, 1, 1)
    dj = jnp.arange(6).reshape(1, 6, 1, 1)
    ei = jnp.arange(4).reshape(1, 1, 4, 1)
    ej = jnp.arange(4).reshape(1, 1, 1, 4)
    i1 = di - ei
    j1 = dj - ej
    valid = (i1 >= 0) & (i1 < 3) & (j1 >= 0) & (j1 < 3)
    idx = jnp.clip(i1, 0, 2) * 3 + jnp.clip(j1, 0, 2)
    w = w1m[idx] * valid[..., None].astype(w1m.dtype)
    return w.reshape(_KT, _K2)


def _build_w2e(w2m):
    ei = jnp.arange(4).reshape(4, 1, 1, 1)
    ej = jnp.arange(4).reshape(1, 4, 1, 1)
    dh = jnp.arange(2).reshape(1, 1, 2, 1)
    dw = jnp.arange(2).reshape(1, 1, 1, 2)
    i2 = ei - dh
    j2 = ej - dw
    valid = (i2 >= 0) & (i2 < 3) & (j2 >= 0) & (j2 < 3)
    idx = jnp.clip(i2, 0, 2) * 3 + jnp.clip(j2, 0, 2)
    w = w2m[idx]
    w = w * valid[..., None, None].astype(w2m.dtype)
    w = w.transpose(0, 1, 4, 2, 3, 5)
    return w.reshape(_K2, _N2)


def kernel(w1m, b1, w2m, b2, fc1_w, fc1_b, fc2_w, fc2_b, x):
    B = x.shape[0]
    p = _build_patches(x)
    b1e = jnp.broadcast_to(b1.reshape(1, 1, 32),
                           (16, 1, 32)).reshape(1, _K2)
    w1e = jnp.concatenate([_build_w1e(w1m), b1e],
                          axis=0).astype(jnp.bfloat16)
    w2e = _build_w2e(w2m).astype(jnp.bfloat16)
    return _conv_stage(p, w1e, w2e, b2.reshape(1, 64),
                       fc1_w.astype(jnp.bfloat16), fc1_b, fc2_w, fc2_b)
